# baseline (speedup 1.0000x reference)
import functools

import jax
import jax.numpy as jnp
from jax import lax
from jax.experimental import pallas as pl
from jax.experimental.pallas import tpu as pltpu


def _mm_relu_body(x_ref, w_ref, b_ref, o_ref):
    o_ref[...] = jax.nn.relu(
        jnp.dot(x_ref[...], w_ref[...], preferred_element_type=jnp.float32)
        + b_ref[...]
    )


def _mm2_relu_body(x_ref, p_ref, ws_ref, wn_ref, b_ref, o_ref):
    acc = jnp.dot(x_ref[...], ws_ref[...], preferred_element_type=jnp.float32)
    acc += jnp.dot(p_ref[...], wn_ref[...], preferred_element_type=jnp.float32)
    o_ref[...] = jax.nn.relu(acc + b_ref[...])


def _rows_mm_relu(xf, W, b, blk):
    M, D = xf.shape
    H = W.shape[1]
    grid = M // blk
    return pl.pallas_call(
        _mm_relu_body,
        grid=(grid,),
        in_specs=[
            pl.BlockSpec((blk, D), lambda i: (i, 0)),
            pl.BlockSpec((D, H), lambda i: (0, 0)),
            pl.BlockSpec((1, H), lambda i: (0, 0)),
        ],
        out_specs=pl.BlockSpec((blk, H), lambda i: (i, 0)),
        out_shape=jax.ShapeDtypeStruct((M, H), jnp.float32),
    )(xf, W, b.reshape(1, H))


def _conv_update(xf, pooled, W_self, W_neigh, b_conv, blk):
    M, D = xf.shape
    H = W_self.shape[1]
    grid = M // blk
    return pl.pallas_call(
        _mm2_relu_body,
        grid=(grid,),
        in_specs=[
            pl.BlockSpec((blk, D), lambda i: (i, 0)),
            pl.BlockSpec((blk, D), lambda i: (i, 0)),
            pl.BlockSpec((D, H), lambda i: (0, 0)),
            pl.BlockSpec((D, H), lambda i: (0, 0)),
            pl.BlockSpec((1, H), lambda i: (0, 0)),
        ],
        out_specs=pl.BlockSpec((blk, H), lambda i: (i, 0)),
        out_shape=jax.ShapeDtypeStruct((M, H), jnp.float32),
    )(xf, pooled, W_self, W_neigh, b_conv.reshape(1, H))


def _heads_body(hb_ref, advw_ref, v1w_ref, advb_ref, v1b_ref,
                v2w_ref, v2b_ref, v3w_ref, v3b_ref, gmean_ref,
                o_ref, adv_acc, val_acc):
    i = pl.program_id(0)
    nsteps = pl.num_programs(0)

    @pl.when(i == 0)
    def _init():
        adv_acc[...] = jnp.zeros_like(adv_acc)
        val_acc[...] = jnp.zeros_like(val_acc)

    hb = hb_ref[...]
    adv_acc[...] += jnp.dot(hb, advw_ref[...], preferred_element_type=jnp.float32)
    val_acc[...] += jnp.dot(hb, v1w_ref[...], preferred_element_type=jnp.float32)

    @pl.when(i == nsteps - 1)
    def _fin():
        adv = jax.nn.relu(adv_acc[...] + advb_ref[...])
        v = jax.nn.relu(val_acc[...] + v1b_ref[...])
        v = jax.nn.relu(
            jnp.dot(v, v2w_ref[...], preferred_element_type=jnp.float32)
            + v2b_ref[...]
        )
        v = (jnp.dot(v, v3w_ref[...], preferred_element_type=jnp.float32)
             + v3b_ref[...])
        advm = jnp.dot(adv, gmean_ref[...], preferred_element_type=jnp.float32)
        o_ref[...] = v + adv - advm


def _heads(hb, adv_W, adv_b, val1_W, val1_b, val2_W, val2_b, val3_W, val3_b,
           n_groups, fblk):
    Bn, F = hb.shape
    A = adv_W.shape[1]
    V = val1_W.shape[1]
    grid = F // fblk
    ga = A // n_groups
    gidx = jnp.arange(A) // ga
    gmean = jnp.where(gidx[:, None] == gidx[None, :], 1.0 / ga, 0.0
                      ).astype(jnp.float32)
    return pl.pallas_call(
        _heads_body,
        grid=(grid,),
        in_specs=[
            pl.BlockSpec((Bn, fblk), lambda i: (0, i)),
            pl.BlockSpec((fblk, A), lambda i: (i, 0)),
            pl.BlockSpec((fblk, V), lambda i: (i, 0)),
            pl.BlockSpec((1, A), lambda i: (0, 0)),
            pl.BlockSpec((1, V), lambda i: (0, 0)),
            pl.BlockSpec((V, V), lambda i: (0, 0)),
            pl.BlockSpec((1, V), lambda i: (0, 0)),
            pl.BlockSpec((V, 1), lambda i: (0, 0)),
            pl.BlockSpec((1, 1), lambda i: (0, 0)),
            pl.BlockSpec((A, A), lambda i: (0, 0)),
        ],
        out_specs=pl.BlockSpec((Bn, A), lambda i: (0, 0)),
        out_shape=jax.ShapeDtypeStruct((Bn, A), jnp.float32),
        scratch_shapes=[
            pltpu.VMEM((Bn, A), jnp.float32),
            pltpu.VMEM((Bn, V), jnp.float32),
        ],
    )(hb, adv_W, val1_W, adv_b.reshape(1, A), val1_b.reshape(1, V),
      val2_W, val2_b.reshape(1, V), val3_W, val3_b.reshape(1, 1), gmean)


def kernel(x, edge_index, W_pool, b_pool, W_self, W_neigh, b_conv,
           adv_W, adv_b, val1_W, val1_b, val2_W, val2_b, val3_W, val3_b):
    Bn, Nn, Dd = x.shape
    Hh = W_self.shape[1]
    NA_groups = 4
    xf = x.reshape(Bn * Nn, Dd)

    m = _rows_mm_relu(xf, W_pool, b_pool, blk=2000)

    offs = jnp.arange(Bn, dtype=edge_index.dtype) * Nn
    src = (edge_index[0][None, :] + offs[:, None]).reshape(-1)
    dst = (edge_index[1][None, :] + offs[:, None]).reshape(-1)
    pooled = jax.ops.segment_max(m[src], dst, num_segments=Bn * Nn)
    pooled = jnp.where(jnp.isfinite(pooled), pooled, 0.0)

    h = _conv_update(xf, pooled, W_self, W_neigh, b_conv, blk=2000)
    hb = h.reshape(Bn, Nn * Hh)

    q32 = _heads(hb, adv_W, adv_b, val1_W, val1_b, val2_W, val2_b,
                 val3_W, val3_b, n_groups=NA_groups, fblk=16000)
    A = adv_W.shape[1]
    return q32.reshape(Bn, NA_groups, A // NA_groups)
